# Initial kernel scaffold; baseline (speedup 1.0000x reference)
import jax
import jax.numpy as jnp
from jax.experimental import pallas as pl


def kernel(feature_indices, feature_values, edge_indices, edge_weights, W1, b1, W2, b2):
    raise NotImplementedError("write your pallas kernel here")


# speedup vs baseline: 11.7047x; 11.7047x over previous
import functools

import jax
import jax.numpy as jnp
from jax import lax
from jax.experimental import pallas as pl
from jax.experimental.pallas import tpu as pltpu
from jax.experimental.pallas import tpu_sc as plsc

N = 10000
NP = 10240
F = 10000
H = 64
HALF = 32
ALPHA = 0.1
ITERS = 10
NC = 2
NS = 16
CH = 128
RPT = NP // NS

NNZ_F = 200000
NCH_F = -(-NNZ_F // (NS * CH))
NNZ_E = 320000
NCH_E = -(-NNZ_E // (NS * CH))

_mesh = plsc.VectorSubcoreMesh(core_axis_name="c", subcore_axis_name="s")
_sc_params = pltpu.CompilerParams(use_tc_tiling_on_sc=False)


def _pad_split(rows, cols, vals, nch):
    tot = NS * nch * CH
    pad = tot - rows.shape[0]
    r = jnp.concatenate([rows, jnp.zeros((pad,), jnp.int32)])
    c = jnp.concatenate([cols, jnp.zeros((pad,), jnp.int32)])
    v = jnp.concatenate([vals, jnp.zeros((pad,), jnp.float32)])
    return (r.reshape(NS, nch, CH), c.reshape(NS, nch * CH),
            v.reshape(NS, nch * CH))


def _scale_chunk(gbuf, w_v, base):
    for b in range(CH // 16):
        w16 = w_v[pl.ds(base + b * 16, 16)]
        for i in range(16):
            e = b * 16 + i
            w = w16[i]
            gbuf[e, pl.ds(0, 16)] = gbuf[e, pl.ds(0, 16)] * w
            gbuf[e, pl.ds(16, 16)] = gbuf[e, pl.ds(16, 16)] * w


@functools.partial(
    pl.kernel,
    out_type=jax.ShapeDtypeStruct((NC, NP, HALF), jnp.float32),
    mesh=_mesh,
    compiler_params=_sc_params,
    scratch_types=[
        pltpu.VMEM((NCH_F, CH), jnp.int32),
        pltpu.VMEM((NCH_F * CH,), jnp.int32),
        pltpu.VMEM((NCH_F * CH,), jnp.float32),
        pltpu.VMEM((CH, HALF), jnp.float32),
        pltpu.VMEM_SHARED((NP, HALF), jnp.float32),
    ],
)
def _feat_spmm(w1f, fr, fc, fv, init, out,
               rows_v, cols_v, vals_v, gbuf, acc):
    c = lax.axis_index("c")
    s = lax.axis_index("s")
    rs = pl.ds(s * RPT, RPT)
    pltpu.sync_copy(fr.at[s], rows_v)
    pltpu.sync_copy(fc.at[c, s], cols_v)
    pltpu.sync_copy(fv.at[s], vals_v)
    pltpu.sync_copy(init.at[c, rs], acc.at[rs])
    plsc.subcore_barrier()

    @pl.loop(0, NCH_F)
    def _(j):
        pltpu.sync_copy(w1f.at[cols_v.at[pl.ds(j * CH, CH)]], gbuf)
        _scale_chunk(gbuf, vals_v, j * CH)
        pltpu.sync_copy(gbuf, acc.at[rows_v.at[j]], add=True)

    plsc.subcore_barrier()
    pltpu.sync_copy(acc.at[rs], out.at[c, rs])


@functools.partial(
    pl.kernel,
    out_type=jax.ShapeDtypeStruct((NC, NP, HALF), jnp.float32),
    mesh=_mesh,
    compiler_params=_sc_params,
    scratch_types=[
        pltpu.VMEM((NCH_E, CH), jnp.int32),
        pltpu.VMEM((NCH_E * CH,), jnp.int32),
        pltpu.VMEM((NCH_E * CH,), jnp.float32),
        pltpu.VMEM((CH, HALF), jnp.float32),
        pltpu.VMEM_SHARED((NP, HALF), jnp.float32),
        pltpu.VMEM_SHARED((NP, HALF), jnp.float32),
    ],
)
def _appnp(er, ec, ew, l2s, als, out,
           rows_v, cols_v, w_v, gbuf, state_a, state_b):
    c = lax.axis_index("c")
    s = lax.axis_index("s")
    rs = pl.ds(s * RPT, RPT)
    pltpu.sync_copy(er.at[s], rows_v)
    pltpu.sync_copy(ec.at[s], cols_v)
    pltpu.sync_copy(ew.at[s], w_v)
    pltpu.sync_copy(l2s.at[c, rs], state_a.at[rs])
    plsc.subcore_barrier()

    def one_iter(src, dst):
        pltpu.sync_copy(als.at[c, rs], dst.at[rs])
        plsc.subcore_barrier()

        @pl.loop(0, NCH_E)
        def _(j):
            pltpu.sync_copy(src.at[cols_v.at[pl.ds(j * CH, CH)]], gbuf)
            _scale_chunk(gbuf, w_v, j * CH)
            pltpu.sync_copy(gbuf, dst.at[rows_v.at[j]], add=True)

        plsc.subcore_barrier()

    @pl.loop(0, ITERS // 2)
    def _(i):
        one_iter(state_a, state_b)
        one_iter(state_b, state_a)

    pltpu.sync_copy(state_a.at[rs], out.at[c, rs])


def _mlp_body(l1s_ref, w2_ref, b2_ref, l2s_ref, als_ref):
    x = jnp.concatenate([l1s_ref[0], l1s_ref[1]], axis=1)
    x = jnp.maximum(x, 0.0)
    y = lax.dot_general(x, w2_ref[...], (((1,), (0,)), ((), ())),
                        preferred_element_type=jnp.float32) + b2_ref[...]
    l2s_ref[0] = y[:, :HALF]
    l2s_ref[1] = y[:, HALF:]
    als_ref[0] = ALPHA * y[:, :HALF]
    als_ref[1] = ALPHA * y[:, HALF:]


_mlp = pl.pallas_call(
    _mlp_body,
    out_shape=[jax.ShapeDtypeStruct((NC, NP, HALF), jnp.float32),
               jax.ShapeDtypeStruct((NC, NP, HALF), jnp.float32)],
)


def _lsm_body(ps_ref, out_ref):
    x = jnp.concatenate([ps_ref[0][:N], ps_ref[1][:N]], axis=1)
    m = jnp.max(x, axis=1, keepdims=True)
    sh = x - m
    out_ref[...] = sh - jnp.log(jnp.sum(jnp.exp(sh), axis=1, keepdims=True))


_lsm = pl.pallas_call(
    _lsm_body,
    out_shape=jax.ShapeDtypeStruct((N, H), jnp.float32),
)


def kernel(feature_indices, feature_values, edge_indices, edge_weights,
           W1, b1, W2, b2):
    fr, fc, fv = _pad_split(feature_indices[0], feature_indices[1],
                            feature_values, NCH_F)
    fc2 = jnp.stack([fc, fc + F])
    w1f = jnp.concatenate([W1[:, :HALF], W1[:, HALF:]], axis=0)
    l1init = jnp.stack([jnp.broadcast_to(b1[:HALF], (NP, HALF)),
                        jnp.broadcast_to(b1[HALF:], (NP, HALF))])
    er, ec, ew = _pad_split(edge_indices[0], edge_indices[1],
                            edge_weights * (1.0 - ALPHA), NCH_E)

    l1s = _feat_spmm(w1f, fr, fc2, fv, l1init)
    l2s, als = _mlp(l1s, W2, b2.reshape(1, H))
    ps = _appnp(er, ec, ew, l2s, als)
    return _lsm(ps)
